# baseline (speedup 1.0000x reference)
import functools
import math

import jax
import jax.numpy as jnp
from jax import lax
from jax.experimental import pallas as pl
from jax.experimental.pallas import tpu as pltpu
from jax.experimental.pallas import tpu_sc as plsc

_D = 8
_B_STATIC = 4
_NB = 512

_NC = 2
_NS = 16
_NW = _NC * _NS

_NOISE_CACHE = {}


def _noise_scaled(m, n):
    key = (m, n)
    if key not in _NOISE_CACHE:
        u = jax.random.uniform(jax.random.key(1), (_B_STATIC, m, n),
                               minval=1e-8, maxval=1.0, dtype=jnp.float32)
        _NOISE_CACHE[key] = jax.block_until_ready(-jnp.log(-jnp.log(u)) / 1000.0)
    return _NOISE_CACHE[key]


def _make_topk_idx_body(m, n, nb):
    def body(q_ref, idx_ref):
        jblk = pl.program_id(0)
        bb = pl.program_id(1)
        work = q_ref[0]
        iota = lax.broadcasted_iota(jnp.int32, (m, nb), 0)
        lane = lax.broadcasted_iota(jnp.int32, (1, nb), 1)
        sentinel = jnp.float32(-1.0)
        col_base = bb * (m * n) + jblk * nb + lane
        for j in range(_D):
            v = jnp.max(work, axis=0, keepdims=True)
            ij = jnp.min(jnp.where(work == v, iota, jnp.int32(m)),
                         axis=0, keepdims=True)
            work = jnp.where(iota == ij, sentinel, work)
            idx_ref[0, j:j + 1, :] = col_base + ij * n
    return body


def _sc_scatter_body(idx_hbm, zeros_hbm, sval_hbm, out_hbm,
                     zbuf, idx_v, val_v, sv16, sem):
    c = lax.axis_index("c")
    s = lax.axis_index("s")
    chunk = idx_v.shape[0]
    total = out_hbm.shape[0]
    slab = total // _NW
    zchunk = zbuf.shape[0]
    n_fill = slab // zchunk

    @pl.when(s == 0)
    def _():
        pltpu.sync_copy(zeros_hbm, zbuf)
    plsc.subcore_barrier()
    slab_base = (c * _NS + s) * slab
    copies = []
    for k in range(n_fill):
        copies.append(pltpu.async_copy(
            zbuf, out_hbm.at[pl.ds(slab_base + k * zchunk, zchunk)], sem))
    for cp in copies:
        cp.wait()
    plsc.subcore_barrier()

    n_idx_half = chunk * _NS
    pltpu.sync_copy(idx_hbm.at[pl.ds(c * n_idx_half + s * chunk, chunk)], idx_v)
    pltpu.sync_copy(sval_hbm, sv16)
    sval = sv16[...]
    for i in range(chunk // 16):
        val_v[pl.ds(i * 16, 16)] = sval
    pltpu.async_copy(val_v, out_hbm.at[idx_v], sem).wait()


def kernel(param, scalar, b):
    m, n = param.shape[1], param.shape[2]
    noise = _noise_scaled(m, n)

    zz = jnp.broadcast_to(param, (_B_STATIC, m, n)) + noise
    mx = jnp.max(zz, axis=1, keepdims=True)
    e = jnp.exp(zz - lax.stop_gradient(mx))
    q = e / jnp.sum(e, axis=1, keepdims=True)

    b_factor = jnp.asarray(b).astype(jnp.float32) / jnp.float32(_B_STATIC)
    s = (jnp.maximum(jnp.float32(0.01), scalar[0]) * b_factor
         / jnp.float32(math.sqrt(_D)))

    nb = min(_NB, n)
    grid = (n // nb, _B_STATIC)
    idx = pl.pallas_call(
        _make_topk_idx_body(m, n, nb),
        grid=grid,
        in_specs=[pl.BlockSpec((1, m, nb), lambda j, bb: (bb, 0, j))],
        out_specs=pl.BlockSpec((1, _D, nb), lambda j, bb: (bb, 0, j)),
        out_shape=jax.ShapeDtypeStruct((_B_STATIC, _D, n), jnp.int32),
    )(q)

    total = _B_STATIC * m * n
    n_idx = _B_STATIC * _D * n
    chunk = n_idx // _NW
    idx_flat = idx.reshape(n_idx)
    zchunk = 65536
    zeros_small = jnp.zeros((zchunk,), jnp.float32)
    sval = jnp.broadcast_to(s, (16,))

    mesh = plsc.VectorSubcoreMesh(core_axis_name="c", subcore_axis_name="s")
    scatter = functools.partial(
        pl.kernel,
        out_type=jax.ShapeDtypeStruct((total,), jnp.float32),
        mesh=mesh,
        scratch_types=[
            pltpu.VMEM_SHARED((zchunk,), jnp.float32),
            pltpu.VMEM((chunk,), jnp.int32),
            pltpu.VMEM((chunk,), jnp.float32),
            pltpu.VMEM((16,), jnp.float32),
            pltpu.SemaphoreType.DMA,
        ],
    )(_sc_scatter_body)
    out_flat = scatter(idx_flat, zeros_small, sval)
    return out_flat.reshape(_B_STATIC, m, n)


# speedup vs baseline: 1.0109x; 1.0109x over previous
import functools
import math

import jax
import jax.numpy as jnp
from jax import lax
from jax.experimental import pallas as pl
from jax.experimental.pallas import tpu as pltpu
from jax.experimental.pallas import tpu_sc as plsc

_D = 8
_B_STATIC = 4
_NB = 512

_NC = 2
_NS = 16
_NW = _NC * _NS

_NOISE_CACHE = {}


def _noise_scaled(m, n):
    key = (m, n)
    if key not in _NOISE_CACHE:
        u = jax.random.uniform(jax.random.key(1), (_B_STATIC, m, n),
                               minval=1e-8, maxval=1.0, dtype=jnp.float32)
        _NOISE_CACHE[key] = jax.block_until_ready(-jnp.log(-jnp.log(u)) / 1000.0)
    return _NOISE_CACHE[key]


def _make_topk_idx_body(m, n, nb):
    def body(q_ref, idx_ref):
        jblk = pl.program_id(0)
        bb = pl.program_id(1)
        work = q_ref[0]
        iota = lax.broadcasted_iota(jnp.int32, (m, nb), 0)
        lane = lax.broadcasted_iota(jnp.int32, (1, nb), 1)
        sentinel = jnp.float32(-1.0)
        col_base = bb * (m * n) + jblk * nb + lane
        for j in range(_D):
            v = jnp.max(work, axis=0, keepdims=True)
            ij = jnp.min(jnp.where(work == v, iota, jnp.int32(m)),
                         axis=0, keepdims=True)
            work = jnp.where(iota == ij, sentinel, work)
            idx_ref[0, j:j + 1, :] = col_base + ij * n
    return body


def _sc_scatter_body(idx_hbm, zeros_hbm, sval_hbm, out_hbm,
                     zbuf, idx_v, val_v, sv16, sem):
    c = lax.axis_index("c")
    s = lax.axis_index("s")
    chunk = idx_v.shape[0]
    total = out_hbm.shape[0]
    slab = total // _NW
    zchunk = zbuf.shape[0]
    n_fill = slab // zchunk

    pltpu.sync_copy(zeros_hbm, zbuf)
    slab_base = (c * _NS + s) * slab
    copies = []
    for k in range(n_fill):
        copies.append(pltpu.async_copy(
            zbuf, out_hbm.at[pl.ds(slab_base + k * zchunk, zchunk)], sem))
    for cp in copies:
        cp.wait()
    plsc.subcore_barrier()

    n_idx_half = chunk * _NS
    pltpu.sync_copy(idx_hbm.at[pl.ds(c * n_idx_half + s * chunk, chunk)], idx_v)
    pltpu.sync_copy(sval_hbm, sv16)
    sval = sv16[...]
    for i in range(chunk // 16):
        val_v[pl.ds(i * 16, 16)] = sval
    pltpu.async_copy(val_v, out_hbm.at[idx_v], sem).wait()


def kernel(param, scalar, b):
    m, n = param.shape[1], param.shape[2]
    noise = _noise_scaled(m, n)

    zz = jnp.broadcast_to(param, (_B_STATIC, m, n)) + noise
    mx = jnp.max(zz, axis=1, keepdims=True)
    e = jnp.exp(zz - lax.stop_gradient(mx))
    q = e / jnp.sum(e, axis=1, keepdims=True)

    b_factor = jnp.asarray(b).astype(jnp.float32) / jnp.float32(_B_STATIC)
    s = (jnp.maximum(jnp.float32(0.01), scalar[0]) * b_factor
         / jnp.float32(math.sqrt(_D)))

    nb = min(_NB, n)
    grid = (n // nb, _B_STATIC)
    idx = pl.pallas_call(
        _make_topk_idx_body(m, n, nb),
        grid=grid,
        in_specs=[pl.BlockSpec((1, m, nb), lambda j, bb: (bb, 0, j))],
        out_specs=pl.BlockSpec((1, _D, nb), lambda j, bb: (bb, 0, j)),
        out_shape=jax.ShapeDtypeStruct((_B_STATIC, _D, n), jnp.int32),
    )(q)

    total = _B_STATIC * m * n
    n_idx = _B_STATIC * _D * n
    chunk = n_idx // _NW
    idx_flat = idx.reshape(n_idx)
    zchunk = 65536
    zeros_small = jnp.zeros((zchunk,), jnp.float32)
    sval = jnp.broadcast_to(s, (16,))

    mesh = plsc.VectorSubcoreMesh(core_axis_name="c", subcore_axis_name="s")
    scatter = functools.partial(
        pl.kernel,
        out_type=jax.ShapeDtypeStruct((total,), jnp.float32),
        mesh=mesh,
        scratch_types=[
            pltpu.VMEM((zchunk,), jnp.float32),
            pltpu.VMEM((chunk,), jnp.int32),
            pltpu.VMEM((chunk,), jnp.float32),
            pltpu.VMEM((16,), jnp.float32),
            pltpu.SemaphoreType.DMA,
        ],
    )(_sc_scatter_body)
    out_flat = scatter(idx_flat, zeros_small, sval)
    return out_flat.reshape(_B_STATIC, m, n)
